# Initial kernel scaffold; baseline (speedup 1.0000x reference)
import jax
import jax.numpy as jnp
from jax.experimental import pallas as pl


def kernel(X):
    raise NotImplementedError("write your pallas kernel here")


# speedup vs baseline: 3.7355x; 3.7355x over previous
import functools

import jax
import jax.numpy as jnp
from jax import lax
from jax.experimental import pallas as pl
from jax.experimental.pallas import tpu as pltpu
from jax.experimental.pallas import tpu_sc as plsc

N = 4096
DIM = 16
RB = 256

NC = 2
NS = 16
L = 16
NW = NC * NS
RPW = N // NW
SC_BATCH = 16
CH = N // L


def _d2_body(x_ref, out_ref):
    i = pl.program_id(0)
    X = x_ref[...]
    xb = x_ref[pl.ds(i * RB, RB), :]
    sq = jnp.sum(X * X, axis=1)
    sqb = jnp.sum(xb * xb, axis=1)
    dot = lax.dot_general(xb, X, (((1,), (1,)), ((), ())),
                          preferred_element_type=jnp.float32)
    d2 = sqb[:, None] + sq[None, :] - 2.0 * dot
    out_ref[...] = jnp.maximum(d2, 0.0)


def _pairwise_d2(X):
    return pl.pallas_call(
        _d2_body,
        grid=(N // RB,),
        in_specs=[pl.BlockSpec((N, DIM), lambda i: (0, 0))],
        out_specs=pl.BlockSpec((RB, N), lambda i: (i, 0)),
        out_shape=jax.ShapeDtypeStruct((N, N), jnp.float32),
    )(X)


def _merge32(A, B, v):
    vs = lax.sort(v)
    lo = jnp.minimum(B, lax.rev(vs, (0,)))
    los = lax.sort(lo)
    rlos = lax.rev(los, (0,))
    A2 = lax.sort(jnp.minimum(A, rlos))
    B2 = lax.sort(jnp.maximum(A, rlos))
    return A2, B2


def _stats_body(d2_hbm, a_hbm, b_hbm, rows_v, a_v, b_v):
    wid = lax.axis_index("s") * NC + lax.axis_index("c")
    base = wid * RPW

    def batch_body(bi, _):
        pltpu.sync_copy(d2_hbm.at[pl.ds(base + bi * SC_BATCH, SC_BATCH), :],
                        rows_v)

        def row_body(r, acc):
            accA, accB = acc
            v0 = lax.sort(rows_v[r, pl.ds(0, L)])
            v1 = lax.sort(rows_v[r, pl.ds(L, L)])
            rv1 = lax.rev(v1, (0,))
            A = lax.sort(jnp.minimum(v0, rv1))
            B = lax.sort(jnp.maximum(v0, rv1))
            t = jnp.broadcast_to(jnp.max(B), (L,))

            def chunk_body(i, carry):
                A, B, t = carry
                v = rows_v[r, pl.ds(i * L, L)]

                def do_merge(args):
                    A, B, _ = args
                    A2, B2 = _merge32(A, B, v)
                    return A2, B2, jnp.broadcast_to(jnp.max(B2), (L,))

                return lax.cond(jnp.any(v < t), do_merge, lambda args: args,
                                (A, B, t))

            A, B, t = lax.fori_loop(2, CH, chunk_body, (A, B, t))
            a15 = jnp.max(A)
            lane = lax.iota(jnp.int32, L)
            b14 = jnp.max(jnp.where(lane < L - 1, B, -1.0))
            accA = jnp.where(lane == r, a15, accA)
            accB = jnp.where(lane == r, b14, accB)
            return accA, accB

        zero = jnp.zeros((L,), jnp.float32)
        accA, accB = lax.fori_loop(0, SC_BATCH, row_body, (zero, zero))
        a_v[pl.ds(bi * SC_BATCH, SC_BATCH)] = accA
        b_v[pl.ds(bi * SC_BATCH, SC_BATCH)] = accB
        return 0

    lax.fori_loop(0, RPW // SC_BATCH, batch_body, 0)
    pltpu.sync_copy(a_v, a_hbm.at[pl.ds(base, RPW)])
    pltpu.sync_copy(b_v, b_hbm.at[pl.ds(base, RPW)])


def _row_stats(d2):
    mesh = plsc.VectorSubcoreMesh(core_axis_name="c", subcore_axis_name="s")
    fn = functools.partial(
        pl.kernel, mesh=mesh,
        out_type=[jax.ShapeDtypeStruct((N,), jnp.float32),
                  jax.ShapeDtypeStruct((N,), jnp.float32)],
        scratch_types=[pltpu.VMEM((SC_BATCH, N), jnp.float32),
                       pltpu.VMEM((RPW,), jnp.float32),
                       pltpu.VMEM((RPW,), jnp.float32)],
        compiler_params=pltpu.CompilerParams(needs_layout_passes=False),
    )(_stats_body)
    return fn(d2)


def _w_body(d2_ref, a_ref, b_ref, out_ref):
    i = pl.program_id(0)
    d2 = d2_ref[...]
    a_full = a_ref[...]
    b_full = b_ref[...]
    a_r = a_ref[pl.ds(i * RB, RB)]
    b_r = b_ref[pl.ds(i * RB, RB)]

    def inv_s2(a):
        s = jnp.maximum(jnp.sqrt(a + 1e-12), 1e-7)
        return 1.0 / (s * s)

    wr = jnp.where(d2 <= b_r[:, None],
                   jnp.exp(-d2 * inv_s2(a_r)[:, None]), 0.0)
    wc = jnp.where(d2 <= b_full[None, :],
                   jnp.exp(-d2 * inv_s2(a_full)[None, :]), 0.0)
    out_ref[...] = 0.5 * (wr + wc)


def _affinity_out(d2, a, b):
    return pl.pallas_call(
        _w_body,
        grid=(N // RB,),
        in_specs=[pl.BlockSpec((RB, N), lambda i: (i, 0)),
                  pl.BlockSpec((N,), lambda i: (0,)),
                  pl.BlockSpec((N,), lambda i: (0,))],
        out_specs=pl.BlockSpec((RB, N), lambda i: (i, 0)),
        out_shape=jax.ShapeDtypeStruct((N, N), jnp.float32),
    )(d2, a, b)


def kernel(X):
    d2 = _pairwise_d2(X)
    a, b = _row_stats(d2)
    return _affinity_out(d2, a, b)


# speedup vs baseline: 6.0971x; 1.6322x over previous
import functools

import jax
import jax.numpy as jnp
from jax import lax
from jax.experimental import pallas as pl
from jax.experimental.pallas import tpu as pltpu
from jax.experimental.pallas import tpu_sc as plsc

N = 4096
DIM = 16
KSEL = 31
RB = 256
NBISECT = 10

NC = 2
NS = 16
L = 16
NW = NC * NS
RPW = N // NW
SCB = 8
NBATCH = RPW // SCB
BIG = 1e30


def _d2_body(x_ref, out_ref, thr_ref):
    i = pl.program_id(0)
    X = x_ref[...]
    xb = x_ref[pl.ds(i * RB, RB), :]
    sq = jnp.sum(X * X, axis=1)
    sqb = jnp.sum(xb * xb, axis=1)
    dot = lax.dot_general(xb, X, (((1,), (1,)), ((), ())),
                          preferred_element_type=jnp.float32)
    d2 = jnp.maximum(sqb[:, None] + sq[None, :] - 2.0 * dot, 0.0)
    out_ref[...] = d2

    hi = (jnp.sqrt(sqb) + jnp.sqrt(jnp.max(sq))) ** 2 + 1.0
    lo = jnp.zeros((RB,), jnp.float32)
    for _ in range(NBISECT):
        mid = 0.5 * (lo + hi)
        c = jnp.sum((d2 < mid[:, None]).astype(jnp.int32), axis=1)
        ge = c >= KSEL
        hi = jnp.where(ge, mid, hi)
        lo = jnp.where(ge, lo, mid)
    thr_ref[...] = hi


def _pairwise_d2(X):
    return pl.pallas_call(
        _d2_body,
        grid=(N // RB,),
        in_specs=[pl.BlockSpec((N, DIM), lambda i: (0, 0))],
        out_specs=[pl.BlockSpec((RB, N), lambda i: (i, 0)),
                   pl.BlockSpec((RB,), lambda i: (i,))],
        out_shape=[jax.ShapeDtypeStruct((N, N), jnp.float32),
                   jax.ShapeDtypeStruct((N,), jnp.float32)],
    )(X)


def _merge32(A, B, v):
    vs = lax.sort(v)
    lo = jnp.minimum(B, lax.rev(vs, (0,)))
    rlos = lax.rev(lax.sort(lo), (0,))
    A2 = lax.sort(jnp.minimum(A, rlos))
    B2 = lax.sort(jnp.maximum(A, rlos))
    return A2, B2


def _filter_row(rows, row0, t, cand_v):
    tv = jnp.broadcast_to(t, (L,))
    big = jnp.full((L,), BIG, jnp.float32)

    def chunk_b(g, carry):
        off, cnt = carry
        v = rows[pl.ds(row0 + g * L, L)]
        mask = v < tv

        def slow(carry):
            off, cnt = carry
            cand_v[pl.ds(off, L)] = jnp.where(mask, v, big)
            return off + L, cnt + mask.astype(jnp.int32)

        return lax.cond(jnp.any(mask), slow, lambda c: c, (off, cnt))

    off, cnt = lax.fori_loop(0, N // L, chunk_b,
                             (0, jnp.zeros((L,), jnp.int32)))
    return off, jnp.sum(cnt)


def _select_stats(cand_v, m):
    inf_v = jnp.full((L,), BIG, jnp.float32)
    cand_v[pl.ds(m, L)] = inf_v
    cand_v[pl.ds(m + L, L)] = inf_v
    v0 = lax.sort(cand_v[pl.ds(0, L)])
    rv1 = lax.rev(lax.sort(cand_v[pl.ds(L, L)]), (0,))
    A = lax.sort(jnp.minimum(v0, rv1))
    B = lax.sort(jnp.maximum(v0, rv1))

    def chunk_body(i, carry):
        A, B = carry
        v = cand_v[pl.ds(i * L, L)]
        return lax.cond(jnp.any(v < jnp.broadcast_to(jnp.max(B), (L,))),
                        lambda ab: _merge32(ab[0], ab[1], v),
                        lambda ab: ab, (A, B))

    nch = (m + L - 1) // L
    A, B = lax.fori_loop(2, nch, chunk_body, (A, B))
    a15 = jnp.max(A)
    lane = lax.iota(jnp.int32, L)
    b14 = jnp.max(jnp.where(lane < L - 1, B, -1.0))
    return a15, b14


def _stats_body(d2_hbm, thr_hbm, a_hbm, b_hbm,
                rows0_v, rows1_v, cand_v, thr_v, a_v, b_v, sem0, sem1):
    wid = lax.axis_index("s") * NC + lax.axis_index("c")
    base = wid * RPW
    pltpu.sync_copy(thr_hbm.at[pl.ds(base, RPW)], thr_v)

    def copy(bi, buf_ref, sem):
        return pltpu.make_async_copy(
            d2_hbm.at[pl.ds((base + bi * SCB) * N, SCB * N)], buf_ref, sem)

    copy(0, rows0_v, sem0).start()

    def half(j, half_idx, buf_ref, acc):
        accA, accB = acc
        lane = lax.iota(jnp.int32, L)
        tvec = thr_v[pl.ds(j * L, L)]

        def row_body(r, acc):
            accA, accB = acc
            li = half_idx * SCB + r
            t = jnp.max(jnp.where(lane == li, tvec, -1.0))
            m, c = _filter_row(buf_ref, r * N, t, cand_v)
            m = lax.cond(c < KSEL,
                         lambda: _filter_row(buf_ref, r * N, BIG, cand_v)[0],
                         lambda: m)
            a15, b14 = _select_stats(cand_v, m)
            accA = jnp.where(lane == li, a15, accA)
            accB = jnp.where(lane == li, b14, accB)
            return accA, accB

        return lax.fori_loop(0, SCB, row_body, (accA, accB))

    def pair_body(j, _):
        zero = jnp.zeros((L,), jnp.float32)
        copy(2 * j, rows0_v, sem0).wait()
        copy(2 * j + 1, rows1_v, sem1).start()
        acc = half(j, 0, rows0_v, (zero, zero))
        copy(2 * j + 1, rows1_v, sem1).wait()

        @pl.when(j + 1 < NBATCH // 2)
        def _():
            copy(2 * j + 2, rows0_v, sem0).start()

        accA, accB = half(j, 1, rows1_v, acc)
        a_v[pl.ds(j * L, L)] = accA
        b_v[pl.ds(j * L, L)] = accB
        return 0

    lax.fori_loop(0, NBATCH // 2, pair_body, 0)
    pltpu.sync_copy(a_v, a_hbm.at[pl.ds(base, RPW)])
    pltpu.sync_copy(b_v, b_hbm.at[pl.ds(base, RPW)])


def _row_stats(d2, thr):
    mesh = plsc.VectorSubcoreMesh(core_axis_name="c", subcore_axis_name="s")
    fn = functools.partial(
        pl.kernel, mesh=mesh,
        out_type=[jax.ShapeDtypeStruct((N,), jnp.float32),
                  jax.ShapeDtypeStruct((N,), jnp.float32)],
        scratch_types=[pltpu.VMEM((SCB * N,), jnp.float32),
                       pltpu.VMEM((SCB * N,), jnp.float32),
                       pltpu.VMEM((N + 2 * L,), jnp.float32),
                       pltpu.VMEM((RPW,), jnp.float32),
                       pltpu.VMEM((RPW,), jnp.float32),
                       pltpu.VMEM((RPW,), jnp.float32),
                       pltpu.SemaphoreType.DMA,
                       pltpu.SemaphoreType.DMA],
        compiler_params=pltpu.CompilerParams(needs_layout_passes=False),
    )(_stats_body)
    return fn(d2.reshape(-1), thr)


def _w_body(d2_ref, a_ref, b_ref, out_ref):
    i = pl.program_id(0)
    d2 = d2_ref[...]
    a_full = a_ref[...]
    b_full = b_ref[...]
    a_r = a_ref[pl.ds(i * RB, RB)]
    b_r = b_ref[pl.ds(i * RB, RB)]

    def inv_s2(a):
        s = jnp.maximum(jnp.sqrt(a + 1e-12), 1e-7)
        return 1.0 / (s * s)

    wr = jnp.where(d2 <= b_r[:, None],
                   jnp.exp(-d2 * inv_s2(a_r)[:, None]), 0.0)
    wc = jnp.where(d2 <= b_full[None, :],
                   jnp.exp(-d2 * inv_s2(a_full)[None, :]), 0.0)
    out_ref[...] = 0.5 * (wr + wc)


def _affinity_out(d2, a, b):
    return pl.pallas_call(
        _w_body,
        grid=(N // RB,),
        in_specs=[pl.BlockSpec((RB, N), lambda i: (i, 0)),
                  pl.BlockSpec((N,), lambda i: (0,)),
                  pl.BlockSpec((N,), lambda i: (0,))],
        out_specs=pl.BlockSpec((RB, N), lambda i: (i, 0)),
        out_shape=jax.ShapeDtypeStruct((N, N), jnp.float32),
    )(d2, a, b)


def kernel(X):
    d2, thr = _pairwise_d2(X)
    a, b = _row_stats(d2, thr)
    return _affinity_out(d2, a, b)


# speedup vs baseline: 11.5141x; 1.8885x over previous
import functools

import jax
import jax.numpy as jnp
from jax import lax
from jax.experimental import pallas as pl
from jax.experimental.pallas import tpu as pltpu
from jax.experimental.pallas import tpu_sc as plsc

N = 4096
DIM = 16
KSEL = 31
RB = 256
NBISECT = 10

NC = 2
NS = 16
L = 16
NW = NC * NS
RPW = N // NW
SCB = 8
NBATCH = RPW // SCB
BIG = 1e30


def _d2_body(x_ref, out_ref, thr_ref):
    i = pl.program_id(0)
    X = x_ref[...]
    xb = x_ref[pl.ds(i * RB, RB), :]
    sq = jnp.sum(X * X, axis=1)
    sqb = jnp.sum(xb * xb, axis=1)
    dot = lax.dot_general(xb, X, (((1,), (1,)), ((), ())),
                          preferred_element_type=jnp.float32)
    d2 = jnp.maximum(sqb[:, None] + sq[None, :] - 2.0 * dot, 0.0)
    out_ref[...] = d2

    hi = (jnp.sqrt(sqb) + jnp.sqrt(jnp.max(sq))) ** 2 + 1.0
    lo = jnp.zeros((RB,), jnp.float32)
    for _ in range(NBISECT):
        mid = 0.5 * (lo + hi)
        c = jnp.sum((d2 < mid[:, None]).astype(jnp.int32), axis=1)
        ge = c >= KSEL
        hi = jnp.where(ge, mid, hi)
        lo = jnp.where(ge, lo, mid)
    thr_ref[...] = hi


def _pairwise_d2(X):
    return pl.pallas_call(
        _d2_body,
        grid=(N // RB,),
        in_specs=[pl.BlockSpec((N, DIM), lambda i: (0, 0))],
        out_specs=[pl.BlockSpec((RB, N), lambda i: (i, 0)),
                   pl.BlockSpec((RB,), lambda i: (i,))],
        out_shape=[jax.ShapeDtypeStruct((N, N), jnp.float32),
                   jax.ShapeDtypeStruct((N,), jnp.float32)],
    )(X)


def _merge32(A, B, v):
    vs = lax.sort(v)
    lo = jnp.minimum(B, lax.rev(vs, (0,)))
    rlos = lax.rev(lax.sort(lo), (0,))
    A2 = lax.sort(jnp.minimum(A, rlos))
    B2 = lax.sort(jnp.maximum(A, rlos))
    return A2, B2


def _filter_row(rows, r, t, cand_v):
    tv = jnp.broadcast_to(t, (L,))
    big = jnp.full((L,), BIG, jnp.float32)

    def group_b(g, carry):
        base = g * (4 * L)
        vs = [rows[r, pl.ds(base + k * L, L)] for k in range(4)]
        mn = jnp.minimum(jnp.minimum(vs[0], vs[1]),
                         jnp.minimum(vs[2], vs[3]))

        def slow(carry):
            for vk in vs:
                mask = vk < tv

                def store(carry):
                    off, cnt = carry
                    cand_v[pl.ds(off, L)] = jnp.where(mask, vk, big)
                    return off + L, cnt + mask.astype(jnp.int32)

                carry = lax.cond(jnp.any(mask), store, lambda c: c, carry)
            return carry

        return lax.cond(jnp.any(mn < tv), slow, lambda c: c, carry)

    off, cnt = lax.fori_loop(0, N // (4 * L), group_b,
                             (0, jnp.zeros((L,), jnp.int32)))
    return off, jnp.sum(cnt)


def _select_stats(cand_v, m):
    inf_v = jnp.full((L,), BIG, jnp.float32)
    cand_v[pl.ds(m, L)] = inf_v
    cand_v[pl.ds(m + L, L)] = inf_v
    v0 = lax.sort(cand_v[pl.ds(0, L)])
    rv1 = lax.rev(lax.sort(cand_v[pl.ds(L, L)]), (0,))
    A = lax.sort(jnp.minimum(v0, rv1))
    B = lax.sort(jnp.maximum(v0, rv1))

    def chunk_body(i, carry):
        v = cand_v[pl.ds(i * L, L)]
        return _merge32(carry[0], carry[1], v)

    nch = (m + L - 1) // L
    A, B = lax.fori_loop(2, nch, chunk_body, (A, B))
    a15 = jnp.max(A)
    lane = lax.iota(jnp.int32, L)
    b14 = jnp.max(jnp.where(lane < L - 1, B, -1.0))
    return a15, b14


def _stats_body(d2_hbm, thr_hbm, a_hbm, b_hbm,
                rows0_v, rows1_v, cand_v, thr_v, a_v, b_v, sem0, sem1):
    wid = lax.axis_index("s") * NC + lax.axis_index("c")
    base = wid * RPW
    pltpu.sync_copy(thr_hbm.at[pl.ds(base, RPW)], thr_v)

    def copy(bi, buf_ref, sem):
        return pltpu.make_async_copy(
            d2_hbm.at[pl.ds(base + bi * SCB, SCB), :], buf_ref, sem)

    copy(0, rows0_v, sem0).start()

    def half(j, half_idx, buf_ref, acc):
        accA, accB = acc
        lane = lax.iota(jnp.int32, L)
        tvec = thr_v[pl.ds(j * L, L)]

        def row_body(r, acc):
            accA, accB = acc
            li = half_idx * SCB + r
            t = jnp.max(jnp.where(lane == li, tvec, -1.0))
            m, c = _filter_row(buf_ref, r, t, cand_v)
            m = lax.cond(c < KSEL,
                         lambda: _filter_row(buf_ref, r, BIG, cand_v)[0],
                         lambda: m)
            a15, b14 = _select_stats(cand_v, m)
            accA = jnp.where(lane == li, a15, accA)
            accB = jnp.where(lane == li, b14, accB)
            return accA, accB

        return lax.fori_loop(0, SCB, row_body, (accA, accB))

    def pair_body(j, _):
        zero = jnp.zeros((L,), jnp.float32)
        copy(2 * j, rows0_v, sem0).wait()
        copy(2 * j + 1, rows1_v, sem1).start()
        acc = half(j, 0, rows0_v, (zero, zero))
        copy(2 * j + 1, rows1_v, sem1).wait()

        @pl.when(j + 1 < NBATCH // 2)
        def _():
            copy(2 * j + 2, rows0_v, sem0).start()

        accA, accB = half(j, 1, rows1_v, acc)
        a_v[pl.ds(j * L, L)] = accA
        b_v[pl.ds(j * L, L)] = accB
        return 0

    lax.fori_loop(0, NBATCH // 2, pair_body, 0)
    pltpu.sync_copy(a_v, a_hbm.at[pl.ds(base, RPW)])
    pltpu.sync_copy(b_v, b_hbm.at[pl.ds(base, RPW)])


def _row_stats(d2, thr):
    mesh = plsc.VectorSubcoreMesh(core_axis_name="c", subcore_axis_name="s")
    fn = functools.partial(
        pl.kernel, mesh=mesh,
        out_type=[jax.ShapeDtypeStruct((N,), jnp.float32),
                  jax.ShapeDtypeStruct((N,), jnp.float32)],
        scratch_types=[pltpu.VMEM((SCB, N), jnp.float32),
                       pltpu.VMEM((SCB, N), jnp.float32),
                       pltpu.VMEM((N + 2 * L,), jnp.float32),
                       pltpu.VMEM((RPW,), jnp.float32),
                       pltpu.VMEM((RPW,), jnp.float32),
                       pltpu.VMEM((RPW,), jnp.float32),
                       pltpu.SemaphoreType.DMA,
                       pltpu.SemaphoreType.DMA],
        compiler_params=pltpu.CompilerParams(needs_layout_passes=False),
    )(_stats_body)
    return fn(d2, thr)


def _w_body(d2_ref, a_ref, b_ref, out_ref):
    i = pl.program_id(0)
    d2 = d2_ref[...]
    a_full = a_ref[...]
    b_full = b_ref[...]
    a_r = a_ref[pl.ds(i * RB, RB)]
    b_r = b_ref[pl.ds(i * RB, RB)]

    def inv_s2(a):
        s = jnp.maximum(jnp.sqrt(a + 1e-12), 1e-7)
        return 1.0 / (s * s)

    wr = jnp.where(d2 <= b_r[:, None],
                   jnp.exp(-d2 * inv_s2(a_r)[:, None]), 0.0)
    wc = jnp.where(d2 <= b_full[None, :],
                   jnp.exp(-d2 * inv_s2(a_full)[None, :]), 0.0)
    out_ref[...] = 0.5 * (wr + wc)


def _affinity_out(d2, a, b):
    return pl.pallas_call(
        _w_body,
        grid=(N // RB,),
        in_specs=[pl.BlockSpec((RB, N), lambda i: (i, 0)),
                  pl.BlockSpec((N,), lambda i: (0,)),
                  pl.BlockSpec((N,), lambda i: (0,))],
        out_specs=pl.BlockSpec((RB, N), lambda i: (i, 0)),
        out_shape=jax.ShapeDtypeStruct((N, N), jnp.float32),
    )(d2, a, b)


def kernel(X):
    d2, thr = _pairwise_d2(X)
    a, b = _row_stats(d2, thr)
    return _affinity_out(d2, a, b)


# speedup vs baseline: 14.7317x; 1.2795x over previous
import functools

import jax
import jax.numpy as jnp
from jax import lax
from jax.experimental import pallas as pl
from jax.experimental.pallas import tpu as pltpu
from jax.experimental.pallas import tpu_sc as plsc

N = 4096
DIM = 16
KSEL = 31
RB = 256
NBISECT = 10

NC = 2
NS = 16
L = 16
NW = NC * NS
RPW = N // NW
SCB = 8
NBATCH = RPW // SCB
BIG = 1e30


def _d2_body(x_ref, out_ref, thr_ref):
    i = pl.program_id(0)
    X = x_ref[...]
    xb = x_ref[pl.ds(i * RB, RB), :]
    sq = jnp.sum(X * X, axis=1)
    sqb = jnp.sum(xb * xb, axis=1)
    dot = lax.dot_general(xb, X, (((1,), (1,)), ((), ())),
                          preferred_element_type=jnp.float32)
    d2 = jnp.maximum(sqb[:, None] + sq[None, :] - 2.0 * dot, 0.0)
    out_ref[...] = d2

    hi = (jnp.sqrt(sqb) + jnp.sqrt(jnp.max(sq))) ** 2 + 1.0
    lo = jnp.zeros((RB,), jnp.float32)
    for _ in range(NBISECT):
        mid = 0.5 * (lo + hi)
        c = jnp.sum((d2 < mid[:, None]).astype(jnp.int32), axis=1)
        ge = c >= KSEL
        hi = jnp.where(ge, mid, hi)
        lo = jnp.where(ge, lo, mid)
    thr_ref[...] = hi


def _pairwise_d2(X):
    return pl.pallas_call(
        _d2_body,
        grid=(N // RB,),
        in_specs=[pl.BlockSpec((N, DIM), lambda i: (0, 0))],
        out_specs=[pl.BlockSpec((RB, N), lambda i: (i, 0)),
                   pl.BlockSpec((RB,), lambda i: (i,))],
        out_shape=[jax.ShapeDtypeStruct((N, N), jnp.float32),
                   jax.ShapeDtypeStruct((N,), jnp.float32)],
    )(X)


def _merge32(A, B, v):
    vs = lax.sort(v)
    lo = jnp.minimum(B, lax.rev(vs, (0,)))
    rlos = lax.rev(lax.sort(lo), (0,))
    A2 = lax.sort(jnp.minimum(A, rlos))
    B2 = lax.sort(jnp.maximum(A, rlos))
    return A2, B2


def _filter_row(rows, r, t, cand_v):
    tv = jnp.broadcast_to(t, (L,))

    def group_b(g, off):
        base = g * (4 * L)
        vs = [rows[r, pl.ds(base + k * L, L)] for k in range(4)]
        mn = jnp.minimum(jnp.minimum(vs[0], vs[1]),
                         jnp.minimum(vs[2], vs[3]))

        def slow(off):
            for vk in vs:
                mask = vk < tv
                plsc.store_compressed(cand_v.at[pl.ds(off, L)], vk, mask=mask)
                off = off + plsc.all_reduce_population_count(mask)[0]
            return off

        has = plsc.all_reduce_population_count(mn < tv)[0] > 0
        return lax.cond(has, slow, lambda o: o, off)

    return lax.fori_loop(0, N // (4 * L), group_b, 0)


def _select_stats(cand_v, m):
    inf_v = jnp.full((L,), BIG, jnp.float32)
    cand_v[pl.ds(m, L)] = inf_v
    cand_v[pl.ds(m + L, L)] = inf_v
    v0 = lax.sort(cand_v[pl.ds(0, L)])
    rv1 = lax.rev(lax.sort(cand_v[pl.ds(L, L)]), (0,))
    A = lax.sort(jnp.minimum(v0, rv1))
    B = lax.sort(jnp.maximum(v0, rv1))

    def chunk_body(i, carry):
        v = cand_v[pl.ds(i * L, L)]
        return _merge32(carry[0], carry[1], v)

    nch = (m + L - 1) // L
    A, B = lax.fori_loop(2, nch, chunk_body, (A, B))
    a15 = jnp.max(A)
    lane = lax.iota(jnp.int32, L)
    b14 = jnp.max(jnp.where(lane < L - 1, B, -1.0))
    return a15, b14


def _stats_body(d2_hbm, thr_hbm, a_hbm, b_hbm,
                rows0_v, rows1_v, cand_v, thr_v, a_v, b_v, sem0, sem1):
    wid = lax.axis_index("s") * NC + lax.axis_index("c")
    base = wid * RPW
    pltpu.sync_copy(thr_hbm.at[pl.ds(base, RPW)], thr_v)

    def copy(bi, buf_ref, sem):
        return pltpu.make_async_copy(
            d2_hbm.at[pl.ds(base + bi * SCB, SCB), :], buf_ref, sem)

    copy(0, rows0_v, sem0).start()

    def half(j, half_idx, buf_ref, acc):
        accA, accB = acc
        lane = lax.iota(jnp.int32, L)
        tvec = thr_v[pl.ds(j * L, L)]

        def row_body(r, acc):
            accA, accB = acc
            li = half_idx * SCB + r
            t = jnp.max(jnp.where(lane == li, tvec, -1.0))
            m = _filter_row(buf_ref, r, t, cand_v)
            m = lax.cond(m < KSEL,
                         lambda: _filter_row(buf_ref, r, BIG, cand_v),
                         lambda: m)
            a15, b14 = _select_stats(cand_v, m)
            accA = jnp.where(lane == li, a15, accA)
            accB = jnp.where(lane == li, b14, accB)
            return accA, accB

        return lax.fori_loop(0, SCB, row_body, (accA, accB))

    def pair_body(j, _):
        zero = jnp.zeros((L,), jnp.float32)
        copy(2 * j, rows0_v, sem0).wait()
        copy(2 * j + 1, rows1_v, sem1).start()
        acc = half(j, 0, rows0_v, (zero, zero))
        copy(2 * j + 1, rows1_v, sem1).wait()

        @pl.when(j + 1 < NBATCH // 2)
        def _():
            copy(2 * j + 2, rows0_v, sem0).start()

        accA, accB = half(j, 1, rows1_v, acc)
        a_v[pl.ds(j * L, L)] = accA
        b_v[pl.ds(j * L, L)] = accB
        return 0

    lax.fori_loop(0, NBATCH // 2, pair_body, 0)
    pltpu.sync_copy(a_v, a_hbm.at[pl.ds(base, RPW)])
    pltpu.sync_copy(b_v, b_hbm.at[pl.ds(base, RPW)])


def _row_stats(d2, thr):
    mesh = plsc.VectorSubcoreMesh(core_axis_name="c", subcore_axis_name="s")
    fn = functools.partial(
        pl.kernel, mesh=mesh,
        out_type=[jax.ShapeDtypeStruct((N,), jnp.float32),
                  jax.ShapeDtypeStruct((N,), jnp.float32)],
        scratch_types=[pltpu.VMEM((SCB, N), jnp.float32),
                       pltpu.VMEM((SCB, N), jnp.float32),
                       pltpu.VMEM((N + 2 * L,), jnp.float32),
                       pltpu.VMEM((RPW,), jnp.float32),
                       pltpu.VMEM((RPW,), jnp.float32),
                       pltpu.VMEM((RPW,), jnp.float32),
                       pltpu.SemaphoreType.DMA,
                       pltpu.SemaphoreType.DMA],
        compiler_params=pltpu.CompilerParams(needs_layout_passes=False),
    )(_stats_body)
    return fn(d2, thr)


def _w_body(d2_ref, a_ref, b_ref, out_ref):
    i = pl.program_id(0)
    d2 = d2_ref[...]
    a_full = a_ref[...]
    b_full = b_ref[...]
    a_r = a_ref[pl.ds(i * RB, RB)]
    b_r = b_ref[pl.ds(i * RB, RB)]

    def inv_s2(a):
        s = jnp.maximum(jnp.sqrt(a + 1e-12), 1e-7)
        return 1.0 / (s * s)

    wr = jnp.where(d2 <= b_r[:, None],
                   jnp.exp(-d2 * inv_s2(a_r)[:, None]), 0.0)
    wc = jnp.where(d2 <= b_full[None, :],
                   jnp.exp(-d2 * inv_s2(a_full)[None, :]), 0.0)
    out_ref[...] = 0.5 * (wr + wc)


def _affinity_out(d2, a, b):
    return pl.pallas_call(
        _w_body,
        grid=(N // RB,),
        in_specs=[pl.BlockSpec((RB, N), lambda i: (i, 0)),
                  pl.BlockSpec((N,), lambda i: (0,)),
                  pl.BlockSpec((N,), lambda i: (0,))],
        out_specs=pl.BlockSpec((RB, N), lambda i: (i, 0)),
        out_shape=jax.ShapeDtypeStruct((N, N), jnp.float32),
    )(d2, a, b)


def kernel(X):
    d2, thr = _pairwise_d2(X)
    a, b = _row_stats(d2, thr)
    return _affinity_out(d2, a, b)
